# baseline (speedup 1.0000x reference)
import functools

import jax
import jax.numpy as jnp
from jax import lax
from jax.experimental import pallas as pl
from jax.experimental.pallas import tpu as pltpu
from jax.experimental.pallas import tpu_sc as plsc

B = 16384
SEQ = 200
D = 32
HID = 64
VOCAB_ROWS = 1000000

NC = 2
NS = 16
NW = NC * NS
BPW = B // NW
CB = 8
TOK = CB * SEQ
IDXW = 100
NSTREAM = TOK // IDXW
NCHUNK = BPW // CB
L = 16


def _sc_pool(ids2d, table):
  mesh = plsc.VectorSubcoreMesh(
      core_axis_name="c", subcore_axis_name="s", num_cores=NC, num_subcores=NS)

  @functools.partial(
      pl.kernel,
      out_type=jax.ShapeDtypeStruct((B, D), jnp.float32),
      mesh=mesh,
      compiler_params=pltpu.CompilerParams(
          use_tc_tiling_on_sc=False, needs_layout_passes=False),
      scratch_types=[
          pltpu.VMEM((2, NSTREAM, IDXW), jnp.int32),
          pltpu.VMEM((2, TOK, D // 2), jnp.int32),
          pltpu.VMEM((BPW, D), jnp.float32),
          pltpu.SemaphoreType.DMA,
          pltpu.SemaphoreType.DMA,
          pltpu.SemaphoreType.DMA,
          pltpu.SemaphoreType.DMA,
      ],
  )
  def body(ids_hbm, tab_hbm, out_hbm, idxb, gbuf, obuf, si0, si1, sg0, sg1):
    wid = lax.axis_index("s") * NC + lax.axis_index("c")
    row0 = wid * (BPW * SEQ // IDXW)
    sem_i = (si0, si1)
    sem_g = (sg0, sg1)
    tab_rows = tab_hbm

    def idx_copy(c, d):
      return pltpu.make_async_copy(
          ids_hbm.at[pl.ds(row0 + c * (TOK // IDXW), NSTREAM)],
          idxb.at[d], sem_i[d])

    def gathers(d):
      return [
          pltpu.make_async_copy(
              tab_rows.at[idxb.at[d, r]],
              gbuf.at[d, pl.ds(r * IDXW, IDXW)],
              sem_g[d],
          )
          for r in range(NSTREAM)
      ]

    cp = idx_copy(0, 0)
    cp.start()
    cp.wait()
    for g in gathers(0):
      g.start()
    idx_copy(1, 1).start()

    def pair_body(c2, carry):
      for d in range(2):
        e = 1 - d
        c = c2 * 2 + d
        @pl.when(c + 1 < NCHUNK)
        def _():
          idx_copy(c + 1, e).wait()
          for g in gathers(e):
            g.start()

        for g in gathers(d):
          g.wait()

        @pl.when(c + 2 < NCHUNK)
        def _():
          idx_copy(c + 2, d).start()

        scale = jnp.full((L,), 1.0 / SEQ, jnp.float32)
        mask = jnp.full((L,), -65536, jnp.int32)
        for b in range(CB):
          base = b * SEQ

          def tok_body(t8, accs):
            a = list(accs)
            t = base + t8 * 8
            for j in range(8):
              w = gbuf[d, t + j, :]
              lo = plsc.bitcast(jnp.left_shift(w, 16), jnp.float32)
              hi = plsc.bitcast(jnp.bitwise_and(w, mask), jnp.float32)
              k = (j % 4) * 2
              a[k] = a[k] + lo
              a[k + 1] = a[k + 1] + hi
            return tuple(a)

          z = jnp.zeros((L,), jnp.float32)
          accs = lax.fori_loop(0, SEQ // 8, tok_body, (z,) * 8)
          s_even = (accs[0] + accs[2]) + (accs[4] + accs[6])
          s_odd = (accs[1] + accs[3]) + (accs[5] + accs[7])
          obuf[c * CB + b, pl.ds(0, L)] = s_even * scale
          obuf[c * CB + b, pl.ds(L, L)] = s_odd * scale
      return carry

    lax.fori_loop(0, NCHUNK // 2, pair_body, 0)
    pltpu.sync_copy(obuf, out_hbm.at[pl.ds(wid * BPW, BPW)])

  return body(ids2d, table)


def _mlp(pooled, W1, b1, Wcat, bcat):
  BM = 2048

  def body(p_ref, w1_ref, b1_ref, wc_ref, bc_ref, o_ref):
    p = p_ref[...]
    h = jnp.dot(p, w1_ref[...], preferred_element_type=jnp.float32)
    h = jnp.maximum(h + b1_ref[...], 0.0)
    o = jnp.dot(h, wc_ref[...], preferred_element_type=jnp.float32)
    o_ref[...] = o + bc_ref[...]

  return pl.pallas_call(
      body,
      grid=(B // BM,),
      in_specs=[
          pl.BlockSpec((BM, D), lambda i: (i, 0)),
          pl.BlockSpec((D, HID), lambda i: (0, 0)),
          pl.BlockSpec((1, HID), lambda i: (0, 0)),
          pl.BlockSpec((HID, 8), lambda i: (0, 0)),
          pl.BlockSpec((1, 8), lambda i: (0, 0)),
      ],
      out_specs=pl.BlockSpec((BM, 8), lambda i: (i, 0)),
      out_shape=jax.ShapeDtypeStruct((B, 8), jnp.float32),
  )(pooled, W1, b1, Wcat, bcat)


def kernel(input_ids, E, W1, b1, Wr, br, Wc, bc):
  ids2d = input_ids.astype(jnp.int32).reshape(B * SEQ // IDXW, IDXW)
  Ebf = E.astype(jnp.bfloat16)
  P = jax.lax.bitcast_convert_type(
      Ebf.reshape(VOCAB_ROWS, D // 2, 2), jnp.int32)
  pooled = _sc_pool(ids2d, P)
  perm = jnp.arange(D).reshape(D // 2, 2).T.reshape(D)
  W1p = W1[perm, :]
  Wcat = jnp.concatenate([Wr, Wc], axis=1)
  bcat = jnp.concatenate([br, bc]).reshape(1, 8)
  out8 = _mlp(pooled, W1p, b1.reshape(1, HID), Wcat, bcat)
  return out8[:, :5], out8[:, 5:]


# speedup vs baseline: 1.0923x; 1.0923x over previous
import functools

import jax
import jax.numpy as jnp
from jax import lax
from jax.experimental import pallas as pl
from jax.experimental.pallas import tpu as pltpu
from jax.experimental.pallas import tpu_sc as plsc

B = 16384
SEQ = 200
D = 32
HID = 64
VOCAB_ROWS = 1000000

NC = 2
NS = 16
NW = NC * NS
BPW = B // NW
CB = 8
TOK = CB * SEQ
IDXW = 100
NSTREAM = TOK // IDXW
NCHUNK = BPW // CB
L = 16


def _sc_pool(ids2d, table):
  mesh = plsc.VectorSubcoreMesh(
      core_axis_name="c", subcore_axis_name="s", num_cores=NC, num_subcores=NS)

  @functools.partial(
      pl.kernel,
      out_type=jax.ShapeDtypeStruct((B, D), jnp.float32),
      mesh=mesh,
      compiler_params=pltpu.CompilerParams(
          use_tc_tiling_on_sc=False, needs_layout_passes=False),
      scratch_types=[
          pltpu.VMEM((2, NSTREAM, IDXW), jnp.int32),
          pltpu.VMEM((2, TOK, D), jnp.bfloat16),
          pltpu.VMEM((BPW, D), jnp.float32),
          pltpu.SemaphoreType.DMA,
          pltpu.SemaphoreType.DMA,
          pltpu.SemaphoreType.DMA,
          pltpu.SemaphoreType.DMA,
      ],
  )
  def body(ids_hbm, tab_hbm, out_hbm, idxb, gbuf, obuf, si0, si1, sg0, sg1):
    wid = lax.axis_index("s") * NC + lax.axis_index("c")
    row0 = wid * (BPW * SEQ // IDXW)
    sem_i = (si0, si1)
    sem_g = (sg0, sg1)
    tab_rows = tab_hbm

    def idx_copy(c, d):
      return pltpu.make_async_copy(
          ids_hbm.at[pl.ds(row0 + c * (TOK // IDXW), NSTREAM)],
          idxb.at[d], sem_i[d])

    def gathers(d):
      return [
          pltpu.make_async_copy(
              tab_rows.at[idxb.at[d, r]],
              gbuf.at[d, pl.ds(r * IDXW, IDXW)],
              sem_g[d],
          )
          for r in range(NSTREAM)
      ]

    cp = idx_copy(0, 0)
    cp.start()
    cp.wait()
    for g in gathers(0):
      g.start()
    idx_copy(1, 1).start()

    def pair_body(c2, carry):
      for d in range(2):
        e = 1 - d
        c = c2 * 2 + d
        @pl.when(c + 1 < NCHUNK)
        def _():
          idx_copy(c + 1, e).wait()
          for g in gathers(e):
            g.start()

        for g in gathers(d):
          g.wait()

        @pl.when(c + 2 < NCHUNK)
        def _():
          idx_copy(c + 2, d).start()

        scale = jnp.full((L,), 1.0 / SEQ, jnp.float32)
        mask = jnp.full((L,), -65536, jnp.int32)
        for b in range(CB):
          base = b * SEQ

          def tok_body(t8, accs):
            a = list(accs)
            t = base + t8 * 8
            for j in range(8):
              w = plsc.bitcast(gbuf[d, t + j, :], jnp.int32)
              lo = plsc.bitcast(jnp.left_shift(w, 16), jnp.float32)
              hi = plsc.bitcast(jnp.bitwise_and(w, mask), jnp.float32)
              k = (j % 4) * 2
              a[k] = a[k] + lo
              a[k + 1] = a[k + 1] + hi
            return tuple(a)

          z = jnp.zeros((L,), jnp.float32)
          accs = lax.fori_loop(0, SEQ // 8, tok_body, (z,) * 8)
          s_even = (accs[0] + accs[2]) + (accs[4] + accs[6])
          s_odd = (accs[1] + accs[3]) + (accs[5] + accs[7])
          obuf[c * CB + b, pl.ds(0, L)] = s_even * scale
          obuf[c * CB + b, pl.ds(L, L)] = s_odd * scale
      return carry

    lax.fori_loop(0, NCHUNK // 2, pair_body, 0)
    pltpu.sync_copy(obuf, out_hbm.at[pl.ds(wid * BPW, BPW)])

  return body(ids2d, table)


def _cast_bf16(E):
  BMP = 20000

  def body(e_ref, o_ref):
    o_ref[...] = e_ref[...].astype(jnp.bfloat16)

  return pl.pallas_call(
      body,
      grid=(VOCAB_ROWS // BMP,),
      in_specs=[pl.BlockSpec((BMP, D), lambda i: (i, 0))],
      out_specs=pl.BlockSpec((BMP, D), lambda i: (i, 0)),
      out_shape=jax.ShapeDtypeStruct((VOCAB_ROWS, D), jnp.bfloat16),
  )(E)


def _mlp(pooled, W1, b1, Wcat, bcat):
  BM = 2048

  def body(p_ref, w1_ref, b1_ref, wc_ref, bc_ref, o_ref):
    p = p_ref[...]
    h = jnp.dot(p, w1_ref[...], preferred_element_type=jnp.float32)
    h = jnp.maximum(h + b1_ref[...], 0.0)
    o = jnp.dot(h, wc_ref[...], preferred_element_type=jnp.float32)
    o_ref[...] = o + bc_ref[...]

  return pl.pallas_call(
      body,
      grid=(B // BM,),
      in_specs=[
          pl.BlockSpec((BM, D), lambda i: (i, 0)),
          pl.BlockSpec((D, HID), lambda i: (0, 0)),
          pl.BlockSpec((1, HID), lambda i: (0, 0)),
          pl.BlockSpec((HID, 8), lambda i: (0, 0)),
          pl.BlockSpec((1, 8), lambda i: (0, 0)),
      ],
      out_specs=pl.BlockSpec((BM, 8), lambda i: (i, 0)),
      out_shape=jax.ShapeDtypeStruct((B, 8), jnp.float32),
  )(pooled, W1, b1, Wcat, bcat)


def kernel(input_ids, E, W1, b1, Wr, br, Wc, bc):
  ids2d = input_ids.astype(jnp.int32).reshape(B * SEQ // IDXW, IDXW)
  pooled = _sc_pool(ids2d, _cast_bf16(E))
  perm = jnp.arange(D).reshape(D // 2, 2).T.reshape(D)
  W1p = W1[perm, :]
  Wcat = jnp.concatenate([Wr, Wc], axis=1)
  bcat = jnp.concatenate([br, bc]).reshape(1, 8)
  out8 = _mlp(pooled, W1p, b1.reshape(1, HID), Wcat, bcat)
  return out8[:, :5], out8[:, 5:]


# speedup vs baseline: 1.6120x; 1.4757x over previous
import functools

import jax
import jax.numpy as jnp
from jax import lax
from jax.experimental import pallas as pl
from jax.experimental.pallas import tpu as pltpu
from jax.experimental.pallas import tpu_sc as plsc

B = 16384
SEQ = 200
D = 32
HID = 64
VOCAB_ROWS = 1000000

NC = 2
NS = 16
NW = NC * NS
BPW = B // NW
CB = 8
TOK = CB * SEQ
IDXW = 100
NSTREAM = TOK // IDXW
NCHUNK = BPW // CB
L = 16


def _sc_pool(ids2d, table):
  mesh = plsc.VectorSubcoreMesh(
      core_axis_name="c", subcore_axis_name="s", num_cores=NC, num_subcores=NS)

  @functools.partial(
      pl.kernel,
      out_type=jax.ShapeDtypeStruct((B, D), jnp.float32),
      mesh=mesh,
      compiler_params=pltpu.CompilerParams(
          use_tc_tiling_on_sc=False, needs_layout_passes=False),
      scratch_types=[
          pltpu.VMEM((2, NSTREAM, IDXW), jnp.int32),
          pltpu.VMEM((2, TOK, D), jnp.float32),
          pltpu.VMEM((BPW, D), jnp.float32),
          pltpu.SemaphoreType.DMA,
          pltpu.SemaphoreType.DMA,
          pltpu.SemaphoreType.DMA,
          pltpu.SemaphoreType.DMA,
      ],
  )
  def body(ids_hbm, tab_hbm, out_hbm, idxb, gbuf, obuf, si0, si1, sg0, sg1):
    wid = lax.axis_index("s") * NC + lax.axis_index("c")
    row0 = wid * (BPW * SEQ // IDXW)
    sem_i = (si0, si1)
    sem_g = (sg0, sg1)
    tab_rows = tab_hbm

    def idx_copy(c, d):
      return pltpu.make_async_copy(
          ids_hbm.at[pl.ds(row0 + c * (TOK // IDXW), NSTREAM)],
          idxb.at[d], sem_i[d])

    def gathers(d):
      return [
          pltpu.make_async_copy(
              tab_rows.at[idxb.at[d, r]],
              gbuf.at[d, pl.ds(r * IDXW, IDXW)],
              sem_g[d],
          )
          for r in range(NSTREAM)
      ]

    cp = idx_copy(0, 0)
    cp.start()
    cp.wait()
    for g in gathers(0):
      g.start()
    idx_copy(1, 1).start()

    def pair_body(c2, carry):
      for d in range(2):
        e = 1 - d
        c = c2 * 2 + d
        @pl.when(c + 1 < NCHUNK)
        def _():
          idx_copy(c + 1, e).wait()
          for g in gathers(e):
            g.start()

        for g in gathers(d):
          g.wait()

        @pl.when(c + 2 < NCHUNK)
        def _():
          idx_copy(c + 2, d).start()

        scale = jnp.full((L,), 1.0 / SEQ, jnp.float32)
        for b in range(CB):
          base = b * SEQ

          def tok_body(t8, accs):
            a = list(accs)
            t = base + t8 * 8
            for j in range(8):
              k = (j % 4) * 2
              a[k] = a[k] + gbuf[d, t + j, pl.ds(0, L)]
              a[k + 1] = a[k + 1] + gbuf[d, t + j, pl.ds(L, L)]
            return tuple(a)

          z = jnp.zeros((L,), jnp.float32)
          accs = lax.fori_loop(0, SEQ // 8, tok_body, (z,) * 8)
          s0 = (accs[0] + accs[2]) + (accs[4] + accs[6])
          s1 = (accs[1] + accs[3]) + (accs[5] + accs[7])
          obuf[c * CB + b, pl.ds(0, L)] = s0 * scale
          obuf[c * CB + b, pl.ds(L, L)] = s1 * scale
      return carry

    lax.fori_loop(0, NCHUNK // 2, pair_body, 0)
    pltpu.sync_copy(obuf, out_hbm.at[pl.ds(wid * BPW, BPW)])

  return body(ids2d, table)


def _compact(E):
  BMP4 = 2000
  QR = VOCAB_ROWS // 4
  nblk = QR // BMP4

  def body(e0, e1, e2, e3, o_ref):
    o_ref[...] = jnp.concatenate(
        [e0[...], e1[...], e2[...], e3[...]], axis=1)

  in_specs = [
      pl.BlockSpec((BMP4, D), lambda j, i=i: (i * nblk + j, 0))
      for i in range(4)
  ]
  return pl.pallas_call(
      body,
      grid=(nblk,),
      in_specs=in_specs,
      out_specs=pl.BlockSpec((BMP4, 4 * D), lambda j: (j, 0)),
      out_shape=jax.ShapeDtypeStruct((QR, 4 * D), jnp.float32),
  )(E, E, E, E)


def _mlp(pooled, W1, b1, Wcat, bcat):
  BM = 2048

  def body(p_ref, w1_ref, b1_ref, wc_ref, bc_ref, o_ref):
    p = p_ref[...]
    h = jnp.dot(p, w1_ref[...], preferred_element_type=jnp.float32)
    h = jnp.maximum(h + b1_ref[...], 0.0)
    o = jnp.dot(h, wc_ref[...], preferred_element_type=jnp.float32)
    o_ref[...] = o + bc_ref[...]

  return pl.pallas_call(
      body,
      grid=(B // BM,),
      in_specs=[
          pl.BlockSpec((BM, D), lambda i: (i, 0)),
          pl.BlockSpec((D, HID), lambda i: (0, 0)),
          pl.BlockSpec((1, HID), lambda i: (0, 0)),
          pl.BlockSpec((HID, 8), lambda i: (0, 0)),
          pl.BlockSpec((1, 8), lambda i: (0, 0)),
      ],
      out_specs=pl.BlockSpec((BM, 8), lambda i: (i, 0)),
      out_shape=jax.ShapeDtypeStruct((B, 8), jnp.float32),
  )(pooled, W1, b1, Wcat, bcat)


def kernel(input_ids, E, W1, b1, Wr, br, Wc, bc):
  ids = input_ids.astype(jnp.int32)
  QR = VOCAB_ROWS // 4
  idsu = 4 * (ids % QR) + ids // QR
  ids2d = idsu.reshape(B * SEQ // IDXW, IDXW)
  pooled = _sc_pool(ids2d, _compact(E).reshape(VOCAB_ROWS, D))
  Wcat = jnp.concatenate([Wr, Wc], axis=1)
  bcat = jnp.concatenate([br, bc]).reshape(1, 8)
  out8 = _mlp(pooled, W1, b1.reshape(1, HID), Wcat, bcat)
  return out8[:, :5], out8[:, 5:]


# speedup vs baseline: 2.2530x; 1.3977x over previous
import functools

import jax
import jax.numpy as jnp
from jax import lax
from jax.experimental import pallas as pl
from jax.experimental.pallas import tpu as pltpu
from jax.experimental.pallas import tpu_sc as plsc

B = 16384
SEQ = 200
D = 32
HID = 64
VOCAB_ROWS = 1000000

NC = 2
NS = 16
NW = NC * NS
BPW = B // NW
CB = 8
TOK = CB * SEQ
IDXW = 100
NSTREAM = TOK // IDXW
NCHUNK = BPW // CB
L = 16


def _sc_pool(ids2d, table):
  mesh = plsc.VectorSubcoreMesh(
      core_axis_name="c", subcore_axis_name="s", num_cores=NC, num_subcores=NS)

  @functools.partial(
      pl.kernel,
      out_type=jax.ShapeDtypeStruct((B, D), jnp.float32),
      mesh=mesh,
      compiler_params=pltpu.CompilerParams(
          use_tc_tiling_on_sc=False, needs_layout_passes=False),
      scratch_types=[
          pltpu.VMEM((2, NSTREAM, IDXW), jnp.int32),
          pltpu.VMEM((2, TOK, D), jnp.float32),
          pltpu.VMEM((BPW, D), jnp.float32),
          pltpu.SemaphoreType.DMA,
          pltpu.SemaphoreType.DMA,
          pltpu.SemaphoreType.DMA,
          pltpu.SemaphoreType.DMA,
      ],
  )
  def body(ids_hbm, tab_hbm, out_hbm, idxb, gbuf, obuf, si0, si1, sg0, sg1):
    wid = lax.axis_index("s") * NC + lax.axis_index("c")
    row0 = wid * (BPW * SEQ // IDXW)
    sem_i = (si0, si1)
    sem_g = (sg0, sg1)
    tab_rows = tab_hbm

    def idx_copy(c, d):
      return pltpu.make_async_copy(
          ids_hbm.at[pl.ds(row0 + c * (TOK // IDXW), NSTREAM)],
          idxb.at[d], sem_i[d])

    def gathers(d):
      return [
          pltpu.make_async_copy(
              tab_rows.at[idxb.at[d, r]],
              gbuf.at[d, pl.ds(r * IDXW, IDXW)],
              sem_g[d],
          )
          for r in range(NSTREAM)
      ]

    cp = idx_copy(0, 0)
    cp.start()
    cp.wait()
    for g in gathers(0):
      g.start()
    idx_copy(1, 1).start()

    def pair_body(c2, carry):
      for d in range(2):
        e = 1 - d
        c = c2 * 2 + d
        @pl.when(c + 1 < NCHUNK)
        def _():
          idx_copy(c + 1, e).wait()
          for g in gathers(e):
            g.start()

        for g in gathers(d):
          g.wait()

        @pl.when(c + 2 < NCHUNK)
        def _():
          idx_copy(c + 2, d).start()

        scale = jnp.full((L,), 1.0 / SEQ, jnp.float32)
        for b in range(CB):
          base = b * SEQ

          def tok_body(t8, accs):
            a = list(accs)
            t = base + t8 * 8
            for j in range(8):
              k = (j % 4) * 2
              a[k] = a[k] + gbuf[d, t + j, pl.ds(0, L)]
              a[k + 1] = a[k + 1] + gbuf[d, t + j, pl.ds(L, L)]
            return tuple(a)

          z = jnp.zeros((L,), jnp.float32)
          accs = lax.fori_loop(0, SEQ // 8, tok_body, (z,) * 8)
          s0 = (accs[0] + accs[2]) + (accs[4] + accs[6])
          s1 = (accs[1] + accs[3]) + (accs[5] + accs[7])
          obuf[c * CB + b, pl.ds(0, L)] = s0 * scale
          obuf[c * CB + b, pl.ds(L, L)] = s1 * scale
      return carry

    lax.fori_loop(0, NCHUNK // 2, pair_body, 0)
    pltpu.sync_copy(obuf, out_hbm.at[pl.ds(wid * BPW, BPW)])

  return body(ids2d, table)


BV = 4096
BV4 = BV // 4
NBLK = -(-VOCAB_ROWS // BV)
VOCAB_PAD = NBLK * BV


def _compact(Et):
  def body(e_ref, o_ref):
    xt = e_ref[...].T
    o_ref[...] = jnp.concatenate(
        [lax.slice(xt, (i * BV4, 0), ((i + 1) * BV4, D)) for i in range(4)],
        axis=1)

  return pl.pallas_call(
      body,
      grid=(NBLK,),
      in_specs=[pl.BlockSpec((D, BV), lambda j: (0, j))],
      out_specs=pl.BlockSpec((BV4, 4 * D), lambda j: (j, 0)),
      out_shape=jax.ShapeDtypeStruct((VOCAB_PAD // 4, 4 * D), jnp.float32),
  )(Et)


def _mlp(pooled, W1, b1, Wcat, bcat):
  BM = 2048

  def body(p_ref, w1_ref, b1_ref, wc_ref, bc_ref, o_ref):
    p = p_ref[...]
    h = jnp.dot(p, w1_ref[...], preferred_element_type=jnp.float32)
    h = jnp.maximum(h + b1_ref[...], 0.0)
    o = jnp.dot(h, wc_ref[...], preferred_element_type=jnp.float32)
    o_ref[...] = o + bc_ref[...]

  return pl.pallas_call(
      body,
      grid=(B // BM,),
      in_specs=[
          pl.BlockSpec((BM, D), lambda i: (i, 0)),
          pl.BlockSpec((D, HID), lambda i: (0, 0)),
          pl.BlockSpec((1, HID), lambda i: (0, 0)),
          pl.BlockSpec((HID, 8), lambda i: (0, 0)),
          pl.BlockSpec((1, 8), lambda i: (0, 0)),
      ],
      out_specs=pl.BlockSpec((BM, 8), lambda i: (i, 0)),
      out_shape=jax.ShapeDtypeStruct((B, 8), jnp.float32),
  )(pooled, W1, b1, Wcat, bcat)


def kernel(input_ids, E, W1, b1, Wr, br, Wc, bc):
  ids = input_ids.astype(jnp.int32)
  l = ids & (BV - 1)
  idsu = (ids - l) + ((l & (BV4 - 1)) << 2) + (l >> 10)
  ids2d = idsu.reshape(B * SEQ // IDXW, IDXW)
  pooled = _sc_pool(ids2d, _compact(E.T).reshape(VOCAB_PAD, D))
  Wcat = jnp.concatenate([Wr, Wc], axis=1)
  bcat = jnp.concatenate([br, bc]).reshape(1, 8)
  out8 = _mlp(pooled, W1, b1.reshape(1, HID), Wcat, bcat)
  return out8[:, :5], out8[:, 5:]


# speedup vs baseline: 2.4438x; 1.0847x over previous
import functools

import jax
import jax.numpy as jnp
from jax import lax
from jax.experimental import pallas as pl
from jax.experimental.pallas import tpu as pltpu
from jax.experimental.pallas import tpu_sc as plsc

B = 16384
SEQ = 200
D = 32
HID = 64
VOCAB_ROWS = 1000000

NC = 2
NS = 16
NW = NC * NS
BPW = B // NW
CB = 8
TOK = CB * SEQ
IDXW = 100
NSTREAM = TOK // IDXW
NCHUNK = BPW // CB
L = 16


def _sc_pool(ids2d, table):
  mesh = plsc.VectorSubcoreMesh(
      core_axis_name="c", subcore_axis_name="s", num_cores=NC, num_subcores=NS)

  @functools.partial(
      pl.kernel,
      out_type=jax.ShapeDtypeStruct((B, D), jnp.float32),
      mesh=mesh,
      compiler_params=pltpu.CompilerParams(
          use_tc_tiling_on_sc=False, needs_layout_passes=False),
      scratch_types=[
          pltpu.VMEM((2, NSTREAM, IDXW), jnp.int32),
          pltpu.VMEM((2, TOK, D), jnp.float32),
          pltpu.VMEM((BPW, D), jnp.float32),
          pltpu.SemaphoreType.DMA,
          pltpu.SemaphoreType.DMA,
          pltpu.SemaphoreType.DMA,
          pltpu.SemaphoreType.DMA,
      ],
  )
  def body(ids_hbm, tab_hbm, out_hbm, idxb, gbuf, obuf, si0, si1, sg0, sg1):
    wid = lax.axis_index("s") * NC + lax.axis_index("c")
    row0 = wid * (BPW * SEQ // IDXW)
    sem_i = (si0, si1)
    sem_g = (sg0, sg1)
    tab_rows = tab_hbm

    def idx_copy(c, d):
      return pltpu.make_async_copy(
          ids_hbm.at[pl.ds(row0 + c * (TOK // IDXW), NSTREAM)],
          idxb.at[d], sem_i[d])

    def gathers(d):
      return [
          pltpu.make_async_copy(
              tab_rows.at[idxb.at[d, r]],
              gbuf.at[d, pl.ds(r * IDXW, IDXW)],
              sem_g[d],
          )
          for r in range(NSTREAM)
      ]

    cp = idx_copy(0, 0)
    cp.start()
    cp.wait()
    for g in gathers(0):
      g.start()
    idx_copy(1, 1).start()

    def pair_body(c2, carry):
      for d in range(2):
        e = 1 - d
        c = c2 * 2 + d
        @pl.when(c + 1 < NCHUNK)
        def _():
          idx_copy(c + 1, e).wait()
          for g in gathers(e):
            g.start()

        for g in gathers(d):
          g.wait()

        @pl.when(c + 2 < NCHUNK)
        def _():
          idx_copy(c + 2, d).start()

        scale = jnp.full((L,), 1.0 / SEQ, jnp.float32)
        for b in range(CB):
          base = b * SEQ

          def tok_body(t8, accs):
            a = list(accs)
            t = base + t8 * 8
            for j in range(8):
              k = (j % 4) * 2
              a[k] = a[k] + gbuf[d, t + j, pl.ds(0, L)]
              a[k + 1] = a[k + 1] + gbuf[d, t + j, pl.ds(L, L)]
            return tuple(a)

          z = jnp.zeros((L,), jnp.float32)
          accs = lax.fori_loop(0, SEQ // 8, tok_body, (z,) * 8)
          s0 = (accs[0] + accs[2]) + (accs[4] + accs[6])
          s1 = (accs[1] + accs[3]) + (accs[5] + accs[7])
          obuf[c * CB + b, pl.ds(0, L)] = s0 * scale
          obuf[c * CB + b, pl.ds(L, L)] = s1 * scale
      return carry

    lax.fori_loop(0, NCHUNK // 2, pair_body, 0)
    pltpu.sync_copy(obuf, out_hbm.at[pl.ds(wid * BPW, BPW)])

  return body(ids2d, table)


BV = 8192
BV4 = BV // 4
NBLK = -(-VOCAB_ROWS // BV)
VOCAB_PAD = NBLK * BV


def _compact(Et):
  def body(e_ref, o_ref):
    x = e_ref[...]
    eye = jnp.eye(D, dtype=jnp.float32)
    xt = lax.dot_general(x, eye, (((0,), (0,)), ((), ())),
                         preferred_element_type=jnp.float32)
    o_ref[...] = jnp.concatenate(
        [lax.slice(xt, (i * BV4, 0), ((i + 1) * BV4, D)) for i in range(4)],
        axis=1)

  return pl.pallas_call(
      body,
      grid=(NBLK,),
      in_specs=[pl.BlockSpec((D, BV), lambda j: (0, j))],
      out_specs=pl.BlockSpec((BV4, 4 * D), lambda j: (j, 0)),
      out_shape=jax.ShapeDtypeStruct((VOCAB_PAD // 4, 4 * D), jnp.float32),
  )(Et)


def _mlp(pooled, W1, b1, Wcat, bcat):
  BM = 2048

  def body(p_ref, w1_ref, b1_ref, wc_ref, bc_ref, o_ref):
    p = p_ref[...]
    h = jnp.dot(p, w1_ref[...], preferred_element_type=jnp.float32)
    h = jnp.maximum(h + b1_ref[...], 0.0)
    o = jnp.dot(h, wc_ref[...], preferred_element_type=jnp.float32)
    o_ref[...] = o + bc_ref[...]

  return pl.pallas_call(
      body,
      grid=(B // BM,),
      in_specs=[
          pl.BlockSpec((BM, D), lambda i: (i, 0)),
          pl.BlockSpec((D, HID), lambda i: (0, 0)),
          pl.BlockSpec((1, HID), lambda i: (0, 0)),
          pl.BlockSpec((HID, 8), lambda i: (0, 0)),
          pl.BlockSpec((1, 8), lambda i: (0, 0)),
      ],
      out_specs=pl.BlockSpec((BM, 8), lambda i: (i, 0)),
      out_shape=jax.ShapeDtypeStruct((B, 8), jnp.float32),
  )(pooled, W1, b1, Wcat, bcat)


def kernel(input_ids, E, W1, b1, Wr, br, Wc, bc):
  ids = input_ids.astype(jnp.int32)
  l = ids & (BV - 1)
  idsu = (ids - l) + ((l & (BV4 - 1)) << 2) + (l >> 11)
  ids2d = idsu.reshape(B * SEQ // IDXW, IDXW)
  pooled = _sc_pool(ids2d, _compact(E.T).reshape(VOCAB_PAD, D))
  Wcat = jnp.concatenate([Wr, Wc], axis=1)
  bcat = jnp.concatenate([br, bc]).reshape(1, 8)
  out8 = _mlp(pooled, W1, b1.reshape(1, HID), Wcat, bcat)
  return out8[:, :5], out8[:, 5:]
